# baseline (speedup 1.0000x reference)
import functools

import jax
import jax.numpy as jnp
from jax import lax
from jax.experimental import pallas as pl
from jax.experimental.pallas import tpu as pltpu
from jax.experimental.pallas import tpu_sc as plsc

N = 131072
K = 64
M = 16
R = 16384
NB = N // R
ALPHA = 0.7
BETA = 1.5

NC = 2
NS = 16
NW = NC * NS
ROWS_W = N // NW
CHUNK = 512
IDXROWS = ROWS_W // 128


def _sc_p1(true_hbm, emb_hbm, acc_out, ebuf, lbuf, acc):
    c = lax.axis_index("c")
    s = lax.axis_index("s")
    wid = c * NS + s
    base = wid * ROWS_W

    zeros = jnp.zeros((16,), jnp.float32)
    ones = jnp.ones((16,), jnp.float32)

    def _zero(i, carry):
        for j in range(K // 16 + 1):
            acc[i, pl.ds(j * 16, 16)] = zeros
        return carry

    lax.fori_loop(0, M, _zero, 0)

    def _group(g, carry):
        labv = lbuf[pl.ds(g * 16, 16)]
        for l in range(16):
            lab = labv[l]
            r = g * 16 + l
            for cg in range(K // 16):
                acc[lab, pl.ds(cg * 16, 16)] += ebuf[r, pl.ds(cg * 16, 16)]
            acc[lab, pl.ds(K, 16)] += ones
        return carry

    for ch in range(ROWS_W // CHUNK):
        pltpu.sync_copy(emb_hbm.at[pl.ds(base + ch * CHUNK, CHUNK)], ebuf)
        pltpu.sync_copy(true_hbm.at[pl.ds(base + ch * CHUNK, CHUNK)], lbuf)
        lax.fori_loop(0, CHUNK // 16, _group, 0)

    pltpu.sync_copy(acc, acc_out.at[wid])


def _sc_pass1(true1, emb2):
    mesh = plsc.VectorSubcoreMesh(core_axis_name="c", subcore_axis_name="s")
    kfn = functools.partial(
        pl.kernel,
        mesh=mesh,
        out_type=jax.ShapeDtypeStruct((NW, M, K + 16), jnp.float32),
        scratch_types=[
            pltpu.VMEM((CHUNK, K), jnp.float32),
            pltpu.VMEM((CHUNK,), jnp.int32),
            pltpu.VMEM((M, K + 16), jnp.float32),
        ],
    )(_sc_p1)
    return kfn(true1, emb2)


def _p2_body(true_l_ref, emb_ref, pts_ref, accs_ref,
             out_ref, mean_s, wrow_s, acc_s):
    step = pl.program_id(0)

    @pl.when(step == 0)
    def _():
        a = accs_ref[0]
        for t in range(1, NW):
            a = a + accs_ref[t]
        seg = a[:, :K]
        cnt_col = a[:, K:K + 1]
        ii = jax.lax.broadcasted_iota(jnp.int32, (M, M), 0)
        jj = jax.lax.broadcasted_iota(jnp.int32, (M, M), 1)
        eye = (ii == jj).astype(jnp.float32)
        cnt_row = jax.lax.dot_general(
            cnt_col, eye, (((0,), (0,)), ((), ())),
            preferred_element_type=jnp.float32)
        mean_s[...] = seg / cnt_col
        lane_ids = jax.lax.broadcasted_iota(jnp.int32, (1, M), 1)
        labmask = (lane_ids >= 1).astype(jnp.float32)
        wrow_s[...] = labmask / cnt_row
        acc_s[...] = jnp.zeros_like(acc_s)
        out_ref[...] = jnp.zeros_like(out_ref)

    lab = true_l_ref[...]
    oh_t = (lab == jax.lax.broadcasted_iota(jnp.int32, (M, 1), 0)
            ).astype(jnp.float32)

    emb = emb_ref[0]
    dt = jax.lax.dot_general(
        mean_s[...], emb, (((1,), (1,)), ((), ())),
        preferred_element_type=jnp.float32)
    dot_own = jnp.sum(oh_t * dt, axis=0, keepdims=True)
    sq = emb * emb
    e2 = jax.lax.dot_general(
        jnp.ones((1, K), jnp.float32), sq, (((1,), (1,)), ((), ())),
        preferred_element_type=jnp.float32)
    m2 = jnp.sum(mean_s[...] * mean_s[...], axis=1, keepdims=True)
    m2_own = jnp.sum(oh_t * m2, axis=0, keepdims=True)
    d2 = jnp.maximum(e2 - 2.0 * dot_own + m2_own, 0.0)
    d = jnp.sqrt(d2)

    pts = pts_ref[0]
    psq = jax.lax.dot_general(
        jnp.ones((1, 3), jnp.float32),
        pts * pts, (((1,), (1,)), ((), ())),
        preferred_element_type=jnp.float32)
    g = jax.nn.sigmoid(jnp.sqrt(psq))

    w = jax.lax.dot_general(
        wrow_s[...], oh_t, (((1,), (0,)), ((), ())),
        preferred_element_type=jnp.float32)
    hinge = jnp.maximum(d - ALPHA, 0.0)
    acc_s[...] += g * hinge * hinge * w

    @pl.when(step == NB - 1)
    def _():
        intra = jnp.sum(acc_s[...])

        m = mean_s[...]
        gram = jax.lax.dot_general(
            m, m, (((1,), (1,)), ((), ())),
            preferred_element_type=jnp.float32)
        ii = jax.lax.broadcasted_iota(jnp.int32, (M, M), 0)
        jj = jax.lax.broadcasted_iota(jnp.int32, (M, M), 1)
        diag = (ii == jj).astype(jnp.float32)
        nrm_col = jnp.sum(gram * diag, axis=1, keepdims=True)
        nrm_row = jnp.sum(gram * diag, axis=0, keepdims=True)
        d2p = jnp.maximum(nrm_col + nrm_row - 2.0 * gram, 0.0)
        dp = jnp.sqrt(d2p)
        hp = jnp.maximum(BETA - dp, 0.0)
        offdiag = ((ii != jj) & (ii >= 1) & (jj >= 1)).astype(jnp.float32)
        inter = jnp.sum(hp * hp * offdiag)

        val = intra / M + inter / (M * (M - 1))
        out_ref[...] = val.reshape(1, 1)


def kernel(points, true, embedding):
    true1 = true.reshape(N)
    emb2 = embedding.reshape(N, K)

    acc_parts = _sc_pass1(true1, emb2)

    out = pl.pallas_call(
        _p2_body,
        grid=(NB,),
        in_specs=[
            pl.BlockSpec((1, R), lambda i: (0, i)),
            pl.BlockSpec((1, R, K), lambda i: (0, i, 0)),
            pl.BlockSpec((1, R, 3), lambda i: (0, i, 0)),
            pl.BlockSpec((NW, M, K + 16), lambda i: (0, 0, 0)),
        ],
        out_specs=pl.BlockSpec((1, 1), lambda i: (0, 0)),
        out_shape=jax.ShapeDtypeStruct((1, 1), jnp.float32),
        scratch_shapes=[
            pltpu.VMEM((M, K), jnp.float32),
            pltpu.VMEM((1, M), jnp.float32),
            pltpu.VMEM((1, R), jnp.float32),
        ],
    )(true, embedding, points, acc_parts)

    return out.reshape(1)


# speedup vs baseline: 1.1217x; 1.1217x over previous
import functools

import jax
import jax.numpy as jnp
from jax import lax
from jax.experimental import pallas as pl
from jax.experimental.pallas import tpu as pltpu
from jax.experimental.pallas import tpu_sc as plsc

N = 131072
K = 64
M = 16
R = 16384
NB = N // R
ALPHA = 0.7
BETA = 1.5

NC = 2
NS = 16
NW = NC * NS
ROWS_W = N // NW
CHUNK = 256
IDXROWS = ROWS_W // 128


W = K + 16


def _sc_p1(true_hbm, emb_hbm, acc_out,
           ebuf0, lbuf0, ebuf1, lbuf1, acc, sems):
    c = lax.axis_index("c")
    s = lax.axis_index("s")
    wid = c * NS + s
    base = wid * ROWS_W

    zeros = jnp.zeros((16,), jnp.float32)
    ones = jnp.ones((16,), jnp.float32)

    def _zero(i, carry):
        for j in range(2 * W // 16):
            acc[i, pl.ds(j * 16, 16)] = zeros
        return carry

    lax.fori_loop(0, M, _zero, 0)

    ebufs = (ebuf0, ebuf1)
    lbufs = (lbuf0, lbuf1)

    def _start(ch):
        b = ch % 2
        he = pltpu.make_async_copy(
            emb_hbm.at[pl.ds(base + ch * CHUNK, CHUNK)], ebufs[b], sems.at[2 * b])
        hl = pltpu.make_async_copy(
            true_hbm.at[pl.ds(base + ch * CHUNK, CHUNK)], lbufs[b],
            sems.at[2 * b + 1])
        he.start()
        hl.start()
        return he, hl

    def _compute(ch):
        b = ch % 2
        ebuf, lbuf = ebufs[b], lbufs[b]

        def _group(g, carry):
            labv = lbuf[pl.ds(g * 16, 16)]
            for l in range(16):
                lab = labv[l]
                r = g * 16 + l
                off = (l % 2) * W
                for cg in range(K // 16):
                    acc[lab, pl.ds(off + cg * 16, 16)] += (
                        ebuf[r, pl.ds(cg * 16, 16)])
                acc[lab, pl.ds(off + K, 16)] += ones
            return carry

        lax.fori_loop(0, CHUNK // 16, _group, 0)

    NCH = ROWS_W // CHUNK
    handles = _start(0)
    for ch in range(NCH):
        nxt = _start(ch + 1) if ch + 1 < NCH else None
        handles[0].wait()
        handles[1].wait()
        _compute(ch)
        handles = nxt

    pltpu.sync_copy(acc, acc_out.at[wid])


def _sc_pass1(true1, emb2):
    mesh = plsc.VectorSubcoreMesh(core_axis_name="c", subcore_axis_name="s")
    kfn = functools.partial(
        pl.kernel,
        mesh=mesh,
        out_type=jax.ShapeDtypeStruct((NW, M, 2 * W), jnp.float32),
        scratch_types=[
            pltpu.VMEM((CHUNK, K), jnp.float32),
            pltpu.VMEM((CHUNK,), jnp.int32),
            pltpu.VMEM((CHUNK, K), jnp.float32),
            pltpu.VMEM((CHUNK,), jnp.int32),
            pltpu.VMEM((M, 2 * W), jnp.float32),
            pltpu.SemaphoreType.DMA((4,)),
        ],
    )(_sc_p1)
    return kfn(true1, emb2)


def _p2_body(true_l_ref, emb_ref, pts_ref, accs_ref,
             out_ref, mean_s, wrow_s, acc_s):
    step = pl.program_id(0)

    @pl.when(step == 0)
    def _():
        a = accs_ref[0]
        for t in range(1, NW):
            a = a + accs_ref[t]
        seg = a[:, :K] + a[:, W:W + K]
        cnt_col = a[:, K:K + 1] + a[:, W + K:W + K + 1]
        ii = jax.lax.broadcasted_iota(jnp.int32, (M, M), 0)
        jj = jax.lax.broadcasted_iota(jnp.int32, (M, M), 1)
        eye = (ii == jj).astype(jnp.float32)
        cnt_row = jax.lax.dot_general(
            cnt_col, eye, (((0,), (0,)), ((), ())),
            preferred_element_type=jnp.float32)
        mean_s[...] = seg / cnt_col
        lane_ids = jax.lax.broadcasted_iota(jnp.int32, (1, M), 1)
        labmask = (lane_ids >= 1).astype(jnp.float32)
        wrow_s[...] = labmask / cnt_row
        acc_s[...] = jnp.zeros_like(acc_s)
        out_ref[...] = jnp.zeros_like(out_ref)

    lab = true_l_ref[...]
    oh_t = (lab == jax.lax.broadcasted_iota(jnp.int32, (M, 1), 0)
            ).astype(jnp.float32)

    emb = emb_ref[0]
    dt = jax.lax.dot_general(
        mean_s[...], emb, (((1,), (1,)), ((), ())),
        preferred_element_type=jnp.float32)
    dot_own = jnp.sum(oh_t * dt, axis=0, keepdims=True)
    sq = emb * emb
    e2 = jax.lax.dot_general(
        jnp.ones((1, K), jnp.float32), sq, (((1,), (1,)), ((), ())),
        preferred_element_type=jnp.float32)
    m2 = jnp.sum(mean_s[...] * mean_s[...], axis=1, keepdims=True)
    m2_own = jnp.sum(oh_t * m2, axis=0, keepdims=True)
    d2 = jnp.maximum(e2 - 2.0 * dot_own + m2_own, 0.0)
    d = jnp.sqrt(d2)

    pts = pts_ref[0]
    psq = jax.lax.dot_general(
        jnp.ones((1, 3), jnp.float32),
        pts * pts, (((1,), (1,)), ((), ())),
        preferred_element_type=jnp.float32)
    g = jax.nn.sigmoid(jnp.sqrt(psq))

    w = jax.lax.dot_general(
        wrow_s[...], oh_t, (((1,), (0,)), ((), ())),
        preferred_element_type=jnp.float32)
    hinge = jnp.maximum(d - ALPHA, 0.0)
    acc_s[...] += g * hinge * hinge * w

    @pl.when(step == NB - 1)
    def _():
        intra = jnp.sum(acc_s[...])

        m = mean_s[...]
        gram = jax.lax.dot_general(
            m, m, (((1,), (1,)), ((), ())),
            preferred_element_type=jnp.float32)
        ii = jax.lax.broadcasted_iota(jnp.int32, (M, M), 0)
        jj = jax.lax.broadcasted_iota(jnp.int32, (M, M), 1)
        diag = (ii == jj).astype(jnp.float32)
        nrm_col = jnp.sum(gram * diag, axis=1, keepdims=True)
        nrm_row = jnp.sum(gram * diag, axis=0, keepdims=True)
        d2p = jnp.maximum(nrm_col + nrm_row - 2.0 * gram, 0.0)
        dp = jnp.sqrt(d2p)
        hp = jnp.maximum(BETA - dp, 0.0)
        offdiag = ((ii != jj) & (ii >= 1) & (jj >= 1)).astype(jnp.float32)
        inter = jnp.sum(hp * hp * offdiag)

        val = intra / M + inter / (M * (M - 1))
        out_ref[...] = val.reshape(1, 1)


def kernel(points, true, embedding):
    true1 = true.reshape(N)
    emb2 = embedding.reshape(N, K)

    acc_parts = _sc_pass1(true1, emb2)

    out = pl.pallas_call(
        _p2_body,
        grid=(NB,),
        in_specs=[
            pl.BlockSpec((1, R), lambda i: (0, i)),
            pl.BlockSpec((1, R, K), lambda i: (0, i, 0)),
            pl.BlockSpec((1, R, 3), lambda i: (0, i, 0)),
            pl.BlockSpec((NW, M, 2 * W), lambda i: (0, 0, 0)),
        ],
        out_specs=pl.BlockSpec((1, 1), lambda i: (0, 0)),
        out_shape=jax.ShapeDtypeStruct((1, 1), jnp.float32),
        scratch_shapes=[
            pltpu.VMEM((M, K), jnp.float32),
            pltpu.VMEM((1, M), jnp.float32),
            pltpu.VMEM((1, R), jnp.float32),
        ],
    )(true, embedding, points, acc_parts)

    return out.reshape(1)


# speedup vs baseline: 1.1896x; 1.0605x over previous
import functools

import jax
import jax.numpy as jnp
from jax import lax
from jax.experimental import pallas as pl
from jax.experimental.pallas import tpu as pltpu
from jax.experimental.pallas import tpu_sc as plsc

N = 131072
K = 64
M = 16
R = 16384
NB = N // R
ALPHA = 0.7
BETA = 1.5

NC = 2
NS = 16
NW = NC * NS
ROWS_W = N // NW
CHUNK = 256
IDXROWS = ROWS_W // 128


W = K + 16


def _sc_p1(true_hbm, emb_hbm, acc_out,
           ebuf0, lbuf0, ebuf1, lbuf1, acc, sems):
    c = lax.axis_index("c")
    s = lax.axis_index("s")
    wid = c * NS + s
    base = wid * ROWS_W

    zeros = jnp.zeros((16,), jnp.float32)
    ones = jnp.ones((16,), jnp.float32)

    def _zero(i, carry):
        for j in range(2 * W // 16):
            acc[i, pl.ds(j * 16, 16)] = zeros
        return carry

    lax.fori_loop(0, M, _zero, 0)

    ebufs = (ebuf0, ebuf1)
    lbufs = (lbuf0, lbuf1)

    def _start(ch):
        b = ch % 2
        he = pltpu.make_async_copy(
            emb_hbm.at[pl.ds(base + ch * CHUNK, CHUNK)], ebufs[b], sems.at[2 * b])
        hl = pltpu.make_async_copy(
            true_hbm.at[pl.ds(base + ch * CHUNK, CHUNK)], lbufs[b],
            sems.at[2 * b + 1])
        he.start()
        hl.start()
        return he, hl

    def _compute(ch):
        b = ch % 2
        ebuf, lbuf = ebufs[b], lbufs[b]

        def _group(g, carry):
            labv = lbuf[pl.ds(g * 16, 16)]
            for l in range(16):
                lab = labv[l]
                r = g * 16 + l
                off = (l % 2) * W
                for cg in range(K // 16):
                    acc[lab, pl.ds(off + cg * 16, 16)] += (
                        ebuf[r, pl.ds(cg * 16, 16)])
                acc[lab, pl.ds(off + K, 16)] += ones
            return carry

        lax.fori_loop(0, CHUNK // 16, _group, 0)

    NCH = ROWS_W // CHUNK
    handles = _start(0)
    for ch in range(NCH):
        nxt = _start(ch + 1) if ch + 1 < NCH else None
        handles[0].wait()
        handles[1].wait()
        _compute(ch)
        handles = nxt

    pltpu.sync_copy(acc, acc_out.at[wid])


def _sc_pass1(true1, emb2):
    mesh = plsc.VectorSubcoreMesh(core_axis_name="c", subcore_axis_name="s")
    kfn = functools.partial(
        pl.kernel,
        mesh=mesh,
        out_type=jax.ShapeDtypeStruct((NW, M, 2 * W), jnp.float32),
        scratch_types=[
            pltpu.VMEM((CHUNK, K), jnp.float32),
            pltpu.VMEM((CHUNK,), jnp.int32),
            pltpu.VMEM((CHUNK, K), jnp.float32),
            pltpu.VMEM((CHUNK,), jnp.int32),
            pltpu.VMEM((M, 2 * W), jnp.float32),
            pltpu.SemaphoreType.DMA((4,)),
        ],
    )(_sc_p1)
    return kfn(true1, emb2)


def _p2_body(true_l_ref, emb_ref, pts_ref, accs_ref,
             out_ref, mean_s, wrow_s, acc_s):
    step = pl.program_id(0)

    @pl.when(step == 0)
    def _():
        a = accs_ref[0]
        for t in range(1, NW):
            a = a + accs_ref[t]
        seg = a[:, :K] + a[:, W:W + K]
        cnt_col = a[:, K:K + 1] + a[:, W + K:W + K + 1]
        ii = jax.lax.broadcasted_iota(jnp.int32, (M, M), 0)
        jj = jax.lax.broadcasted_iota(jnp.int32, (M, M), 1)
        eye = (ii == jj).astype(jnp.float32)
        cnt_row = jax.lax.dot_general(
            cnt_col, eye, (((0,), (0,)), ((), ())),
            preferred_element_type=jnp.float32)
        mean_s[...] = seg / cnt_col
        lane_ids = jax.lax.broadcasted_iota(jnp.int32, (1, M), 1)
        labmask = (lane_ids >= 1).astype(jnp.float32)
        wrow_s[...] = labmask / cnt_row
        acc_s[...] = jnp.zeros_like(acc_s)
        out_ref[...] = jnp.zeros_like(out_ref)

    lab = true_l_ref[...]
    oh_t = (lab == jax.lax.broadcasted_iota(jnp.int32, (M, 1), 0)
            ).astype(jnp.float32)

    emb = emb_ref[...]
    dt = jax.lax.dot_general(
        mean_s[...], emb, (((1,), (1,)), ((), ())),
        preferred_element_type=jnp.float32)
    dot_own = jnp.sum(oh_t * dt, axis=0, keepdims=True)
    sq = emb * emb
    e2 = jax.lax.dot_general(
        jnp.ones((1, K), jnp.float32), sq, (((1,), (1,)), ((), ())),
        preferred_element_type=jnp.float32)
    m2 = jnp.sum(mean_s[...] * mean_s[...], axis=1, keepdims=True)
    m2_own = jnp.sum(oh_t * m2, axis=0, keepdims=True)
    d2 = jnp.maximum(e2 - 2.0 * dot_own + m2_own, 0.0)
    d = jnp.sqrt(d2)

    pts = pts_ref[...]
    psq = jax.lax.dot_general(
        jnp.ones((1, 3), jnp.float32),
        pts * pts, (((1,), (1,)), ((), ())),
        preferred_element_type=jnp.float32)
    g = jax.nn.sigmoid(jnp.sqrt(psq))

    w = jax.lax.dot_general(
        wrow_s[...], oh_t, (((1,), (0,)), ((), ())),
        preferred_element_type=jnp.float32)
    hinge = jnp.maximum(d - ALPHA, 0.0)
    acc_s[...] += g * hinge * hinge * w

    @pl.when(step == NB - 1)
    def _():
        intra = jnp.sum(acc_s[...])

        m = mean_s[...]
        gram = jax.lax.dot_general(
            m, m, (((1,), (1,)), ((), ())),
            preferred_element_type=jnp.float32)
        ii = jax.lax.broadcasted_iota(jnp.int32, (M, M), 0)
        jj = jax.lax.broadcasted_iota(jnp.int32, (M, M), 1)
        diag = (ii == jj).astype(jnp.float32)
        nrm_col = jnp.sum(gram * diag, axis=1, keepdims=True)
        nrm_row = jnp.sum(gram * diag, axis=0, keepdims=True)
        d2p = jnp.maximum(nrm_col + nrm_row - 2.0 * gram, 0.0)
        dp = jnp.sqrt(d2p)
        hp = jnp.maximum(BETA - dp, 0.0)
        offdiag = ((ii != jj) & (ii >= 1) & (jj >= 1)).astype(jnp.float32)
        inter = jnp.sum(hp * hp * offdiag)

        val = intra / M + inter / (M * (M - 1))
        out_ref[...] = val.reshape(1, 1)


def kernel(points, true, embedding):
    true1 = true.reshape(N)
    emb2 = embedding.reshape(N, K)

    acc_parts = _sc_pass1(true1, emb2)

    out = pl.pallas_call(
        _p2_body,
        grid=(NB,),
        in_specs=[
            pl.BlockSpec((1, R), lambda i: (0, i)),
            pl.BlockSpec((R, K), lambda i: (i, 0)),
            pl.BlockSpec((R, 3), lambda i: (i, 0)),
            pl.BlockSpec((NW, M, 2 * W), lambda i: (0, 0, 0)),
        ],
        out_specs=pl.BlockSpec((1, 1), lambda i: (0, 0)),
        out_shape=jax.ShapeDtypeStruct((1, 1), jnp.float32),
        scratch_shapes=[
            pltpu.VMEM((M, K), jnp.float32),
            pltpu.VMEM((1, M), jnp.float32),
            pltpu.VMEM((1, R), jnp.float32),
        ],
    )(true, emb2, points.reshape(N, 3), acc_parts)

    return out.reshape(1)
